# baseline (speedup 1.0000x reference)
import jax
import jax.numpy as jnp
from jax.experimental import pallas as pl

NUM_LAYERS = 3


def _mean4_kernel(a_ref, b_ref, c_ref, d_ref, o_ref):
    o_ref[...] = 0.25 * (a_ref[...] + b_ref[...] + c_ref[...] + d_ref[...])


def _lgconv(x, src, dst, norm):
    msg = norm[:, None] * jnp.take(x, src, axis=0)
    return jnp.zeros_like(x).at[dst].add(msg)


def kernel(edge_index, feats_tensor, emb_table, W, b):
    src = edge_index[0]
    dst = edge_index[1]
    n = emb_table.shape[0]
    deg = jnp.zeros((n,), dtype=jnp.float32).at[dst].add(1.0)
    dinv = jnp.where(deg > 0, jax.lax.rsqrt(jnp.where(deg > 0, deg, 1.0)), 0.0)
    norm = dinv[src] * dinv[dst]
    emb = emb_table + feats_tensor @ W.T + b
    embs = [emb]
    for _ in range(NUM_LAYERS):
        emb = _lgconv(emb, src, dst, norm)
        embs.append(emb)
    blk = pl.BlockSpec((5000, 64), lambda i: (i, 0))
    out = pl.pallas_call(
        _mean4_kernel,
        grid=(10,),
        in_specs=[blk] * 4,
        out_specs=blk,
        out_shape=jax.ShapeDtypeStruct(emb.shape, emb.dtype),
    )(*embs)
    return out


# speedup vs baseline: 8.8405x; 8.8405x over previous
import jax
import jax.numpy as jnp
from jax import lax
from jax.experimental import pallas as pl
from jax.experimental.pallas import tpu as pltpu
from jax.experimental.pallas import tpu_sc as plsc

N = 50000
E = 800000
D = 64
H = 32
FEAT = 128
NUM_LAYERS = 3

NT = 16
NP = 50176
STRIPE = NP // NT
CR = 112
NCR = STRIPE // CR
EP = 802816
ET = EP // NT
SB = 128
NSB = 2
CH = SB * NSB
NCH = ET // CH


def _rsqrt_newton(x):
    i = lax.bitcast_convert_type(x, jnp.int32)
    i = jnp.int32(0x5F3759DF) - lax.shift_right_arithmetic(i, 1)
    y = lax.bitcast_convert_type(i, jnp.float32)
    for _ in range(3):
        y = y * (1.5 - 0.5 * x * y * y)
    return y


def _sc_body(src_hbm, dst_hbm, emb0_hbm,
             sum_hbm, xp_hbm,
             srci, dsti, rows_v, ones_v, a_v, s_v, d_v, zbuf,
             acc_sh, deg_sh, dinv_sh, gsem):
    c = lax.axis_index("c")
    s = lax.axis_index("s")
    ebase = s * ET
    r0 = s * STRIPE
    emb0_c = emb0_hbm.at[c]
    sum_c = sum_hbm.at[c]
    xp_c = xp_hbm.at[c]

    z16 = jnp.zeros((16,), jnp.float32)
    ones16 = jnp.full((16,), 1.0, jnp.float32)

    def expand_mul(buf):
        def row(r, _):
            ids = jnp.full((16,), r, jnp.int32)
            dval = plsc.load_gather(d_v, [ids])
            buf[r, pl.ds(0, 16)] = buf[r, pl.ds(0, 16)] * dval
            buf[r, pl.ds(16, 16)] = buf[r, pl.ds(16, 16)] * dval
            return _
        lax.fori_loop(0, CR, row, 0)

    def zrow(r, _):
        zbuf[r, pl.ds(0, 16)] = z16
        zbuf[r, pl.ds(16, 16)] = z16
        return _
    lax.fori_loop(0, CR, zrow, 0)
    for g in range(CR // 16):
        d_v[pl.ds(16 * g, 16)] = z16
    for g in range(SB // 16):
        ones_v[pl.ds(16 * g, 16)] = ones16
    def zchunk(j, _):
        rr = r0 + j * CR
        pltpu.sync_copy(zbuf, acc_sh.at[pl.ds(rr, CR)])
        pltpu.sync_copy(d_v, deg_sh.at[pl.ds(rr, CR)])
        return _
    lax.fori_loop(0, NCR, zchunk, 0)
    plsc.subcore_barrier()

    def deg_chunk(kc, _):
        base = ebase + kc * CH
        for j in range(NSB):
            pltpu.sync_copy(dst_hbm.at[pl.ds(base + SB * j, SB)], dsti.at[j])
        for j in range(NSB):
            pltpu.sync_copy(ones_v, deg_sh.at[dsti.at[j]], add=True)
        return _
    lax.fori_loop(0, NCH, deg_chunk, 0)
    plsc.subcore_barrier()

    def dinv_chunk(j, _):
        rr = r0 + j * CR
        pltpu.sync_copy(deg_sh.at[pl.ds(rr, CR)], d_v)
        def grp(g, _):
            x = d_v[pl.ds(16 * g, 16)]
            m = x > 0.5
            y = _rsqrt_newton(jnp.where(m, x, 1.0))
            d_v[pl.ds(16 * g, 16)] = jnp.where(m, y, 0.0)
            return _
        lax.fori_loop(0, CR // 16, grp, 0)
        pltpu.sync_copy(d_v, dinv_sh.at[pl.ds(rr, CR)])
        return _
    lax.fori_loop(0, NCR, dinv_chunk, 0)
    plsc.subcore_barrier()

    def xp0_chunk(j, _):
        rr = r0 + j * CR
        pltpu.sync_copy(emb0_c.at[pl.ds(rr, CR)], a_v)
        pltpu.sync_copy(dinv_sh.at[pl.ds(rr, CR)], d_v)
        expand_mul(a_v)
        pltpu.sync_copy(a_v, xp_c.at[pl.ds(rr, CR)])
        return _
    lax.fori_loop(0, NCR, xp0_chunk, 0)
    plsc.subcore_barrier()

    for layer in range(1, NUM_LAYERS + 1):
        def edge_chunk(kc, _):
            base = ebase + kc * CH
            for j in range(NSB):
                pltpu.sync_copy(src_hbm.at[pl.ds(base + SB * j, SB)],
                                srci.at[j])
                pltpu.sync_copy(dst_hbm.at[pl.ds(base + SB * j, SB)],
                                dsti.at[j])
            handles = []
            for j in range(NSB):
                handles.append(pltpu.async_copy(
                    xp_c.at[srci.at[j]],
                    rows_v.at[pl.ds(SB * j, SB)], gsem))
            for h in handles:
                h.wait()
            for j in range(NSB):
                pltpu.sync_copy(rows_v.at[pl.ds(SB * j, SB)],
                                acc_sh.at[dsti.at[j]], add=True)
            return _
        lax.fori_loop(0, NCH, edge_chunk, 0)
        plsc.subcore_barrier()

        def post_chunk(j, _):
            rr = r0 + j * CR
            pltpu.sync_copy(acc_sh.at[pl.ds(rr, CR)], a_v)
            pltpu.sync_copy(zbuf, acc_sh.at[pl.ds(rr, CR)])
            pltpu.sync_copy(dinv_sh.at[pl.ds(rr, CR)], d_v)
            expand_mul(a_v)
            if layer == 1:
                pltpu.sync_copy(a_v, sum_c.at[pl.ds(rr, CR)])
            else:
                pltpu.sync_copy(sum_c.at[pl.ds(rr, CR)], s_v)
                def addrow(r, _):
                    s_v[r, pl.ds(0, 16)] = (s_v[r, pl.ds(0, 16)]
                                            + a_v[r, pl.ds(0, 16)])
                    s_v[r, pl.ds(16, 16)] = (s_v[r, pl.ds(16, 16)]
                                             + a_v[r, pl.ds(16, 16)])
                    return _
                lax.fori_loop(0, CR, addrow, 0)
                pltpu.sync_copy(s_v, sum_c.at[pl.ds(rr, CR)])
            if layer < NUM_LAYERS:
                expand_mul(a_v)
                pltpu.sync_copy(a_v, xp_c.at[pl.ds(rr, CR)])
            return _
        lax.fori_loop(0, NCR, post_chunk, 0)
        plsc.subcore_barrier()


def _sc_pass(src, dst, emb0_halves):
    mesh = plsc.VectorSubcoreMesh(core_axis_name="c", subcore_axis_name="s",
                                  num_cores=2, num_subcores=NT)
    f = pl.kernel(
        _sc_body,
        mesh=mesh,
        compiler_params=pltpu.CompilerParams(needs_layout_passes=False,
                                             use_tc_tiling_on_sc=False),
        out_type=[
            jax.ShapeDtypeStruct((2, NP, H), jnp.float32),
            jax.ShapeDtypeStruct((2, NP, H), jnp.float32),
        ],
        scratch_types=[
            pltpu.VMEM((NSB, SB), jnp.int32),
            pltpu.VMEM((NSB, SB), jnp.int32),
            pltpu.VMEM((CH, H), jnp.float32),
            pltpu.VMEM((SB,), jnp.float32),
            pltpu.VMEM((CR, H), jnp.float32),
            pltpu.VMEM((CR, H), jnp.float32),
            pltpu.VMEM((CR,), jnp.float32),
            pltpu.VMEM((CR, H), jnp.float32),
            pltpu.VMEM_SHARED((NP, H), jnp.float32),
            pltpu.VMEM_SHARED((NP,), jnp.float32),
            pltpu.VMEM_SHARED((NP,), jnp.float32),
            pltpu.SemaphoreType.DMA,
        ],
    )
    return f(src, dst, emb0_halves)


def _prologue_body(f_ref, w_ref, b_ref, e_ref, o_ref):
    res = (lax.dot_general(f_ref[...], w_ref[...],
                           (((1,), (1,)), ((), ())),
                           preferred_element_type=jnp.float32)
           + b_ref[...] + e_ref[...])
    o_ref[0] = res[:, :H]
    o_ref[1] = res[:, H:]


def _prologue(feats_p, W, b2, emb_p):
    nblk = NP // STRIPE
    return pl.pallas_call(
        _prologue_body,
        grid=(nblk,),
        in_specs=[
            pl.BlockSpec((STRIPE, FEAT), lambda i: (i, 0)),
            pl.BlockSpec((D, FEAT), lambda i: (0, 0)),
            pl.BlockSpec((1, D), lambda i: (0, 0)),
            pl.BlockSpec((STRIPE, D), lambda i: (i, 0)),
        ],
        out_specs=pl.BlockSpec((2, STRIPE, H), lambda i: (0, i, 0)),
        out_shape=jax.ShapeDtypeStruct((2, NP, H), jnp.float32),
    )(feats_p, W, b2, emb_p)


def _epilogue_body(e_ref, s_ref, o_ref):
    o_ref[:, :H] = 0.25 * (e_ref[0] + s_ref[0])
    o_ref[:, H:] = 0.25 * (e_ref[1] + s_ref[1])


def _epilogue(emb0_halves, sum_halves):
    BR = 5000
    e0 = emb0_halves[:, :N, :]
    s0 = sum_halves[:, :N, :]
    return pl.pallas_call(
        _epilogue_body,
        grid=(N // BR,),
        in_specs=[
            pl.BlockSpec((2, BR, H), lambda i: (0, i, 0)),
            pl.BlockSpec((2, BR, H), lambda i: (0, i, 0)),
        ],
        out_specs=pl.BlockSpec((BR, D), lambda i: (i, 0)),
        out_shape=jax.ShapeDtypeStruct((N, D), jnp.float32),
    )(e0, s0)


def kernel(edge_index, feats_tensor, emb_table, W, b):
    pad_ids = (N + (jnp.arange(EP - E, dtype=jnp.int32) % (NP - N)))
    src = jnp.concatenate([edge_index[0], pad_ids])
    dst = jnp.concatenate([edge_index[1], pad_ids])
    feats_p = jnp.pad(feats_tensor, ((0, NP - N), (0, 0)))
    emb_p = jnp.pad(emb_table, ((0, NP - N), (0, 0)))
    b2 = b.reshape(1, D)

    emb0_halves = _prologue(feats_p, W, b2, emb_p)
    sum_halves, _ = _sc_pass(src, dst, emb0_halves)
    return _epilogue(emb0_halves, sum_halves)


# speedup vs baseline: 18.0021x; 2.0363x over previous
import jax
import jax.numpy as jnp
from jax import lax
from jax.experimental import pallas as pl
from jax.experimental.pallas import tpu as pltpu
from jax.experimental.pallas import tpu_sc as plsc

N = 50000
E = 800000
D = 64
H = 32
FEAT = 128
NUM_LAYERS = 3

NT = 16
NP = 50176
STRIPE = NP // NT
CR = 64
NCR = STRIPE // CR
EP = 802816
ET = EP // NT
SB = 128
NB = 4
CHG = SB * NB
NCHG = ET // CHG


def _rsqrt_newton(x):
    i = lax.bitcast_convert_type(x, jnp.int32)
    i = jnp.int32(0x5F3759DF) - lax.shift_right_arithmetic(i, 1)
    y = lax.bitcast_convert_type(i, jnp.float32)
    for _ in range(3):
        y = y * (1.5 - 0.5 * x * y * y)
    return y


def _sc_body(src_hbm, dst_hbm, emb0_hbm,
             sum_hbm, xp_hbm,
             srci, dsti, rows_v, ones_v, a_v, s_v, d_v, zbuf,
             acc_sh, deg_sh, sem_si, sem_di, sem_g, sem_sc):
    c = lax.axis_index("c")
    s = lax.axis_index("s")
    ebase = s * ET
    r0 = s * STRIPE
    emb0_c = emb0_hbm.at[c]
    sum_c = sum_hbm.at[c]
    xp_c = xp_hbm.at[c]

    z16 = jnp.zeros((16,), jnp.float32)
    ones16 = jnp.full((16,), 1.0, jnp.float32)

    def expand_mul(buf):
        def row(r, _):
            ids = jnp.full((16,), r, jnp.int32)
            dval = plsc.load_gather(d_v, [ids])
            buf[r, pl.ds(0, 16)] = buf[r, pl.ds(0, 16)] * dval
            buf[r, pl.ds(16, 16)] = buf[r, pl.ds(16, 16)] * dval
            return _
        lax.fori_loop(0, CR, row, 0)

    def zrow(r, _):
        zbuf[r, pl.ds(0, 16)] = z16
        zbuf[r, pl.ds(16, 16)] = z16
        return _
    lax.fori_loop(0, CR, zrow, 0)
    for g in range(CR // 16):
        d_v[pl.ds(16 * g, 16)] = z16
    for g in range(SB // 16):
        ones_v[pl.ds(16 * g, 16)] = ones16
    def zchunk(j, _):
        rr = r0 + j * CR
        pltpu.sync_copy(zbuf, acc_sh.at[pl.ds(rr, CR)])
        pltpu.sync_copy(d_v, deg_sh.at[pl.ds(rr, CR)])
        return _
    lax.fori_loop(0, NCR, zchunk, 0)
    plsc.subcore_barrier()

    def deg_group(g, _):
        base = ebase + g * CHG
        hd = [pltpu.async_copy(dst_hbm.at[pl.ds(base + SB * b, SB)],
                               dsti.at[b], sem_di) for b in range(NB)]
        hc = []
        for b in range(NB):
            hd[b].wait()
            hc.append(pltpu.async_copy(ones_v, deg_sh.at[dsti.at[b]],
                                       sem_sc, add=True))
        for h in hc:
            h.wait()
        return _
    lax.fori_loop(0, NCHG, deg_group, 0)
    plsc.subcore_barrier()

    def dinv_chunk(j, _):
        rr = r0 + j * CR
        pltpu.sync_copy(deg_sh.at[pl.ds(rr, CR)], d_v)
        def grp(g, _):
            x = d_v[pl.ds(16 * g, 16)]
            m = x > 0.5
            y = _rsqrt_newton(jnp.where(m, x, 1.0))
            d_v[pl.ds(16 * g, 16)] = jnp.where(m, y, 0.0)
            return _
        lax.fori_loop(0, CR // 16, grp, 0)
        pltpu.sync_copy(d_v, deg_sh.at[pl.ds(rr, CR)])
        return _
    lax.fori_loop(0, NCR, dinv_chunk, 0)
    plsc.subcore_barrier()

    def xp0_chunk(j, _):
        rr = r0 + j * CR
        pltpu.sync_copy(emb0_c.at[pl.ds(rr, CR)], a_v)
        pltpu.sync_copy(deg_sh.at[pl.ds(rr, CR)], d_v)
        expand_mul(a_v)
        pltpu.sync_copy(a_v, xp_c.at[pl.ds(rr, CR)])
        return _
    lax.fori_loop(0, NCR, xp0_chunk, 0)
    plsc.subcore_barrier()

    for layer in range(1, NUM_LAYERS + 1):
        def edge_group(g, _):
            base = ebase + g * CHG
            hs = [pltpu.async_copy(src_hbm.at[pl.ds(base + SB * b, SB)],
                                   srci.at[b], sem_si) for b in range(NB)]
            hd = [pltpu.async_copy(dst_hbm.at[pl.ds(base + SB * b, SB)],
                                   dsti.at[b], sem_di) for b in range(NB)]
            hg = []
            for b in range(NB):
                hs[b].wait()
                hg.append(pltpu.async_copy(xp_c.at[srci.at[b]],
                                           rows_v.at[b], sem_g))
            hc = []
            for b in range(NB):
                hg[b].wait()
                hd[b].wait()
                hc.append(pltpu.async_copy(rows_v.at[b],
                                           acc_sh.at[dsti.at[b]],
                                           sem_sc, add=True))
            for h in hc:
                h.wait()
            return _
        lax.fori_loop(0, NCHG, edge_group, 0)
        plsc.subcore_barrier()

        def post_chunk(j, _):
            rr = r0 + j * CR
            pltpu.sync_copy(acc_sh.at[pl.ds(rr, CR)], a_v)
            pltpu.sync_copy(zbuf, acc_sh.at[pl.ds(rr, CR)])
            pltpu.sync_copy(deg_sh.at[pl.ds(rr, CR)], d_v)
            expand_mul(a_v)
            if layer == 1:
                pltpu.sync_copy(a_v, sum_c.at[pl.ds(rr, CR)])
            else:
                pltpu.sync_copy(sum_c.at[pl.ds(rr, CR)], s_v)
                def addrow(r, _):
                    s_v[r, pl.ds(0, 16)] = (s_v[r, pl.ds(0, 16)]
                                            + a_v[r, pl.ds(0, 16)])
                    s_v[r, pl.ds(16, 16)] = (s_v[r, pl.ds(16, 16)]
                                             + a_v[r, pl.ds(16, 16)])
                    return _
                lax.fori_loop(0, CR, addrow, 0)
                pltpu.sync_copy(s_v, sum_c.at[pl.ds(rr, CR)])
            if layer < NUM_LAYERS:
                expand_mul(a_v)
                pltpu.sync_copy(a_v, xp_c.at[pl.ds(rr, CR)])
            return _
        lax.fori_loop(0, NCR, post_chunk, 0)
        plsc.subcore_barrier()


def _sc_pass(src, dst, emb0_halves):
    mesh = plsc.VectorSubcoreMesh(core_axis_name="c", subcore_axis_name="s",
                                  num_cores=2, num_subcores=NT)
    f = pl.kernel(
        _sc_body,
        mesh=mesh,
        compiler_params=pltpu.CompilerParams(needs_layout_passes=False,
                                             use_tc_tiling_on_sc=False),
        out_type=[
            jax.ShapeDtypeStruct((2, NP, H), jnp.float32),
            jax.ShapeDtypeStruct((2, NP, H), jnp.float32),
        ],
        scratch_types=[
            pltpu.VMEM((NB, SB), jnp.int32),
            pltpu.VMEM((NB, SB), jnp.int32),
            pltpu.VMEM((NB, SB, H), jnp.float32),
            pltpu.VMEM((SB,), jnp.float32),
            pltpu.VMEM((CR, H), jnp.float32),
            pltpu.VMEM((CR, H), jnp.float32),
            pltpu.VMEM((CR,), jnp.float32),
            pltpu.VMEM((CR, H), jnp.float32),
            pltpu.VMEM_SHARED((NP, H), jnp.float32),
            pltpu.VMEM_SHARED((NP,), jnp.float32),
            pltpu.SemaphoreType.DMA,
            pltpu.SemaphoreType.DMA,
            pltpu.SemaphoreType.DMA,
            pltpu.SemaphoreType.DMA,
        ],
    )
    return f(src, dst, emb0_halves)


def _prologue_body(f_ref, w_ref, b_ref, e_ref, o_ref):
    res = (lax.dot_general(f_ref[...], w_ref[...],
                           (((1,), (1,)), ((), ())),
                           preferred_element_type=jnp.float32)
           + b_ref[...] + e_ref[...])
    o_ref[0] = res[:, :H]
    o_ref[1] = res[:, H:]


def _prologue(feats_p, W, b2, emb_p):
    nblk = NP // STRIPE
    return pl.pallas_call(
        _prologue_body,
        grid=(nblk,),
        in_specs=[
            pl.BlockSpec((STRIPE, FEAT), lambda i: (i, 0)),
            pl.BlockSpec((D, FEAT), lambda i: (0, 0)),
            pl.BlockSpec((1, D), lambda i: (0, 0)),
            pl.BlockSpec((STRIPE, D), lambda i: (i, 0)),
        ],
        out_specs=pl.BlockSpec((2, STRIPE, H), lambda i: (0, i, 0)),
        out_shape=jax.ShapeDtypeStruct((2, NP, H), jnp.float32),
    )(feats_p, W, b2, emb_p)


def _epilogue_body(e_ref, s_ref, o_ref):
    o_ref[:, :H] = 0.25 * (e_ref[0] + s_ref[0])
    o_ref[:, H:] = 0.25 * (e_ref[1] + s_ref[1])


def _epilogue(emb0_halves, sum_halves):
    BR = 5000
    e0 = emb0_halves[:, :N, :]
    s0 = sum_halves[:, :N, :]
    return pl.pallas_call(
        _epilogue_body,
        grid=(N // BR,),
        in_specs=[
            pl.BlockSpec((2, BR, H), lambda i: (0, i, 0)),
            pl.BlockSpec((2, BR, H), lambda i: (0, i, 0)),
        ],
        out_specs=pl.BlockSpec((BR, D), lambda i: (i, 0)),
        out_shape=jax.ShapeDtypeStruct((N, D), jnp.float32),
    )(e0, s0)


def kernel(edge_index, feats_tensor, emb_table, W, b):
    pad_ids = (N + (jnp.arange(EP - E, dtype=jnp.int32) % (NP - N)))
    src = jnp.concatenate([edge_index[0], pad_ids])
    dst = jnp.concatenate([edge_index[1], pad_ids])
    feats_p = jnp.pad(feats_tensor, ((0, NP - N), (0, 0)))
    emb_p = jnp.pad(emb_table, ((0, NP - N), (0, 0)))
    b2 = b.reshape(1, D)

    emb0_halves = _prologue(feats_p, W, b2, emb_p)
    sum_halves, _ = _sc_pass(src, dst, emb0_halves)
    return _epilogue(emb0_halves, sum_halves)


# speedup vs baseline: 19.2963x; 1.0719x over previous
import jax
import jax.numpy as jnp
from jax import lax
from jax.experimental import pallas as pl
from jax.experimental.pallas import tpu as pltpu
from jax.experimental.pallas import tpu_sc as plsc

N = 50000
E = 800000
D = 64
H = 32
FEAT = 128
NUM_LAYERS = 3

NT = 16
NP = 50176
STRIPE = NP // NT
CR = 64
NCR = STRIPE // CR
ET = E // NT
SB = 112
NB = 5
CHG = SB * NB
NCHG = ET // CHG
TAIL = ET - NCHG * CHG
T2 = TAIL - SB


def _rsqrt_newton(x):
    i = lax.bitcast_convert_type(x, jnp.int32)
    i = jnp.int32(0x5F3759DF) - lax.shift_right_arithmetic(i, 1)
    y = lax.bitcast_convert_type(i, jnp.float32)
    for _ in range(3):
        y = y * (1.5 - 0.5 * x * y * y)
    return y


def _sc_body(edge_hbm, emb0_hbm,
             sum_hbm, xp_hbm,
             srci, dsti, rows_v, ones_v, a_v, s_v, d_v, zbuf,
             tsrc, tdst, trows,
             acc_sh, deg_sh, sem_si, sem_di, sem_g, sem_sc):
    c = lax.axis_index("c")
    s = lax.axis_index("s")
    ebase = s * ET
    tbase = ebase + NCHG * CHG
    r0 = s * STRIPE
    emb0_c = emb0_hbm.at[c]
    sum_c = sum_hbm.at[c]
    xp_c = xp_hbm.at[c]

    z16 = jnp.zeros((16,), jnp.float32)
    ones16 = jnp.full((16,), 1.0, jnp.float32)

    def expand_mul(buf):
        def row(r, _):
            ids = jnp.full((16,), r, jnp.int32)
            dval = plsc.load_gather(d_v, [ids])
            buf[r, pl.ds(0, 16)] = buf[r, pl.ds(0, 16)] * dval
            buf[r, pl.ds(16, 16)] = buf[r, pl.ds(16, 16)] * dval
            return _
        lax.fori_loop(0, CR, row, 0)

    def zrow(r, _):
        zbuf[r, pl.ds(0, 16)] = z16
        zbuf[r, pl.ds(16, 16)] = z16
        return _
    lax.fori_loop(0, CR, zrow, 0)
    for g in range(CR // 16):
        d_v[pl.ds(16 * g, 16)] = z16
    for g in range(SB // 16):
        ones_v[pl.ds(16 * g, 16)] = ones16
    if SB % 16:
        ones_v[pl.ds(SB - 16, 16)] = ones16
    def zchunk(j, _):
        rr = r0 + j * CR
        pltpu.sync_copy(zbuf, acc_sh.at[pl.ds(rr, CR)])
        pltpu.sync_copy(d_v, deg_sh.at[pl.ds(rr, CR)])
        return _
    lax.fori_loop(0, NCR, zchunk, 0)
    plsc.subcore_barrier()


    def deg_group(g, _):
        base = ebase + g * CHG
        hd = [pltpu.async_copy(edge_hbm.at[1, pl.ds(base + SB * b, SB)],
                               dsti.at[b], sem_di) for b in range(NB)]
        hc = []
        for b in range(NB):
            hd[b].wait()
            hc.append(pltpu.async_copy(ones_v, deg_sh.at[dsti.at[b]],
                                       sem_sc, add=True))
        for h in hc:
            h.wait()
        return _
    lax.fori_loop(0, NCHG, deg_group, 0)
    h1 = pltpu.async_copy(edge_hbm.at[1, pl.ds(tbase, SB)], dsti.at[0],
                          sem_di)
    h2 = pltpu.async_copy(edge_hbm.at[1, pl.ds(tbase + SB, T2)], tdst,
                          sem_di)
    h1.wait()
    c1 = pltpu.async_copy(ones_v, deg_sh.at[dsti.at[0]], sem_sc, add=True)
    h2.wait()
    c2 = pltpu.async_copy(ones_v.at[pl.ds(0, T2)], deg_sh.at[tdst],
                          sem_sc, add=True)
    c1.wait()
    c2.wait()
    plsc.subcore_barrier()


    def dinv_chunk(j, _):
        rr = r0 + j * CR
        pltpu.sync_copy(deg_sh.at[pl.ds(rr, CR)], d_v)
        def grp(g, _):
            x = d_v[pl.ds(16 * g, 16)]
            m = x > 0.5
            y = _rsqrt_newton(jnp.where(m, x, 1.0))
            d_v[pl.ds(16 * g, 16)] = jnp.where(m, y, 0.0)
            return _
        lax.fori_loop(0, CR // 16, grp, 0)
        pltpu.sync_copy(d_v, deg_sh.at[pl.ds(rr, CR)])
        return _
    lax.fori_loop(0, NCR, dinv_chunk, 0)
    plsc.subcore_barrier()


    def xp0_chunk(j, _):
        rr = r0 + j * CR
        pltpu.sync_copy(emb0_c.at[pl.ds(rr, CR)], a_v)
        pltpu.sync_copy(deg_sh.at[pl.ds(rr, CR)], d_v)
        expand_mul(a_v)
        pltpu.sync_copy(a_v, xp_c.at[pl.ds(rr, CR)])
        return _
    lax.fori_loop(0, NCR, xp0_chunk, 0)
    plsc.subcore_barrier()


    for layer in range(1, NUM_LAYERS + 1):
        def edge_group(g, _):
            base = ebase + g * CHG
            hs = [pltpu.async_copy(edge_hbm.at[0, pl.ds(base + SB * b, SB)],
                                   srci.at[b], sem_si) for b in range(NB)]
            hd = [pltpu.async_copy(edge_hbm.at[1, pl.ds(base + SB * b, SB)],
                                   dsti.at[b], sem_di) for b in range(NB)]
            hg = []
            for b in range(NB):
                hs[b].wait()
                hg.append(pltpu.async_copy(xp_c.at[srci.at[b]],
                                           rows_v.at[b], sem_g))
            hc = []
            for b in range(NB):
                hg[b].wait()
                hd[b].wait()
                hc.append(pltpu.async_copy(rows_v.at[b],
                                           acc_sh.at[dsti.at[b]],
                                           sem_sc, add=True))
            for h in hc:
                h.wait()
            return _
        lax.fori_loop(0, NCHG, edge_group, 0)
        hs1 = pltpu.async_copy(edge_hbm.at[0, pl.ds(tbase, SB)],
                               srci.at[0], sem_si)
        hs2 = pltpu.async_copy(edge_hbm.at[0, pl.ds(tbase + SB, T2)],
                               tsrc, sem_si)
        hd1 = pltpu.async_copy(edge_hbm.at[1, pl.ds(tbase, SB)],
                               dsti.at[0], sem_di)
        hd2 = pltpu.async_copy(edge_hbm.at[1, pl.ds(tbase + SB, T2)],
                               tdst, sem_di)
        hs1.wait()
        g1 = pltpu.async_copy(xp_c.at[srci.at[0]], rows_v.at[0], sem_g)
        hs2.wait()
        g2 = pltpu.async_copy(xp_c.at[tsrc], trows, sem_g)
        g1.wait()
        hd1.wait()
        c1 = pltpu.async_copy(rows_v.at[0], acc_sh.at[dsti.at[0]],
                              sem_sc, add=True)
        g2.wait()
        hd2.wait()
        c2 = pltpu.async_copy(trows, acc_sh.at[tdst], sem_sc, add=True)
        c1.wait()
        c2.wait()
        plsc.subcore_barrier()


        def post_chunk(j, _):
            rr = r0 + j * CR
            pltpu.sync_copy(acc_sh.at[pl.ds(rr, CR)], a_v)
            if layer < NUM_LAYERS:
                pltpu.sync_copy(zbuf, acc_sh.at[pl.ds(rr, CR)])
            if layer == 1:
                pltpu.sync_copy(a_v, sum_c.at[pl.ds(rr, CR)])
            else:
                pltpu.sync_copy(sum_c.at[pl.ds(rr, CR)], s_v)
                def addrow(r, _):
                    s_v[r, pl.ds(0, 16)] = (s_v[r, pl.ds(0, 16)]
                                            + a_v[r, pl.ds(0, 16)])
                    s_v[r, pl.ds(16, 16)] = (s_v[r, pl.ds(16, 16)]
                                             + a_v[r, pl.ds(16, 16)])
                    return _
                lax.fori_loop(0, CR, addrow, 0)
                if layer < NUM_LAYERS:
                    pltpu.sync_copy(s_v, sum_c.at[pl.ds(rr, CR)])
            if layer < NUM_LAYERS:
                pltpu.sync_copy(deg_sh.at[pl.ds(rr, CR)], d_v)
                def sq(g, _):
                    x = d_v[pl.ds(16 * g, 16)]
                    d_v[pl.ds(16 * g, 16)] = x * x
                    return _
                lax.fori_loop(0, CR // 16, sq, 0)
                expand_mul(a_v)
                pltpu.sync_copy(a_v, xp_c.at[pl.ds(rr, CR)])
            else:
                pltpu.sync_copy(deg_sh.at[pl.ds(rr, CR)], d_v)
                expand_mul(s_v)
                pltpu.sync_copy(emb0_c.at[pl.ds(rr, CR)], a_v)
                def finrow(r, _):
                    s_v[r, pl.ds(0, 16)] = 0.25 * (s_v[r, pl.ds(0, 16)]
                                                   + a_v[r, pl.ds(0, 16)])
                    s_v[r, pl.ds(16, 16)] = 0.25 * (s_v[r, pl.ds(16, 16)]
                                                    + a_v[r, pl.ds(16, 16)])
                    return _
                lax.fori_loop(0, CR, finrow, 0)
                pltpu.sync_copy(s_v, sum_c.at[pl.ds(rr, CR)])
            return _
        lax.fori_loop(0, NCR, post_chunk, 0)
        plsc.subcore_barrier()


def _sc_pass(edge_index, emb0_halves):
    mesh = plsc.VectorSubcoreMesh(core_axis_name="c", subcore_axis_name="s",
                                  num_cores=2, num_subcores=NT)
    f = pl.kernel(
        _sc_body,
        mesh=mesh,
        compiler_params=pltpu.CompilerParams(needs_layout_passes=False,
                                             use_tc_tiling_on_sc=False),
        out_type=[
            jax.ShapeDtypeStruct((2, NP, H), jnp.float32),
            jax.ShapeDtypeStruct((2, NP, H), jnp.float32),
        ],
        scratch_types=[
            pltpu.VMEM((NB, SB), jnp.int32),
            pltpu.VMEM((NB, SB), jnp.int32),
            pltpu.VMEM((NB, SB, H), jnp.float32),
            pltpu.VMEM((SB,), jnp.float32),
            pltpu.VMEM((CR, H), jnp.float32),
            pltpu.VMEM((CR, H), jnp.float32),
            pltpu.VMEM((CR,), jnp.float32),
            pltpu.VMEM((CR, H), jnp.float32),
            pltpu.VMEM((T2,), jnp.int32),
            pltpu.VMEM((T2,), jnp.int32),
            pltpu.VMEM((T2, H), jnp.float32),
            pltpu.VMEM_SHARED((NP, H), jnp.float32),
            pltpu.VMEM_SHARED((NP,), jnp.float32),
            pltpu.SemaphoreType.DMA,
            pltpu.SemaphoreType.DMA,
            pltpu.SemaphoreType.DMA,
            pltpu.SemaphoreType.DMA,
        ],
    )
    return f(edge_index, emb0_halves)


def _prologue_body(f_ref, w_ref, b_ref, e_ref, o_ref):
    res = (lax.dot_general(f_ref[...], w_ref[...],
                           (((1,), (1,)), ((), ())),
                           preferred_element_type=jnp.float32)
           + b_ref[...] + e_ref[...])
    o_ref[0] = res[:, :H]
    o_ref[1] = res[:, H:]


def _prologue(feats_p, W, b2, emb_p):
    nblk = NP // STRIPE
    return pl.pallas_call(
        _prologue_body,
        grid=(nblk,),
        in_specs=[
            pl.BlockSpec((STRIPE, FEAT), lambda i: (i, 0)),
            pl.BlockSpec((D, FEAT), lambda i: (0, 0)),
            pl.BlockSpec((1, D), lambda i: (0, 0)),
            pl.BlockSpec((STRIPE, D), lambda i: (i, 0)),
        ],
        out_specs=pl.BlockSpec((2, STRIPE, H), lambda i: (0, i, 0)),
        out_shape=jax.ShapeDtypeStruct((2, NP, H), jnp.float32),
    )(feats_p, W, b2, emb_p)


def kernel(edge_index, feats_tensor, emb_table, W, b):
    feats_p = jnp.pad(feats_tensor, ((0, NP - N), (0, 0)))
    emb_p = jnp.pad(emb_table, ((0, NP - N), (0, 0)))
    b2 = b.reshape(1, D)

    emb0_halves = _prologue(feats_p, W, b2, emb_p)
    final_halves, _ = _sc_pass(edge_index, emb0_halves)
    return jnp.concatenate([final_halves[0, :N, :],
                            final_halves[1, :N, :]], axis=1)
